# baseline (speedup 1.0000x reference)
import functools

import jax
import jax.numpy as jnp
from jax import lax
from jax.experimental import pallas as pl
from jax.experimental.pallas import tpu as pltpu
from jax.experimental.pallas import tpu_sc as plsc

NC = 2
NS = 16
LANES = 16
NW = NC * NS


def _lse_pad_body(t_ref, lse_ref, tp_ref):
    x = t_ref[...]
    m = jnp.max(x, axis=1, keepdims=True)
    s = jnp.sum(jnp.exp(x - m), axis=1, keepdims=True)
    lse_ref[...] = m + jnp.log(s)
    tp_ref[...] = jnp.pad(x, ((0, 0), (0, tp_ref.shape[1] - x.shape[1])))


def _loss_body(p_ref, o_ref, *, inv_n):
    o_ref[...] = jnp.full((1, 1), inv_n, jnp.float32) * jnp.sum(p_ref[...])


def _sc_gather_fn(V, C, CP, N, CH):
    b_per_w = N // NW
    n_ch = b_per_w // CH
    n_full = C // 128
    tail = C - n_full * 128
    tail0 = n_full * 128
    mesh = plsc.VectorSubcoreMesh(core_axis_name="c", subcore_axis_name="s")

    def body(table_h, idx_h, tgt_h, lse_h, out_h, part_h,
             idx_v, tgt_v, lse_v, rows_v, tails_v, acc_v,
             gsem0, gsem1, osem0, osem1):
        cid = lax.axis_index("c")
        sid = lax.axis_index("s")
        wid = sid * NC + cid
        base = wid * b_per_w
        pltpu.sync_copy(idx_h.at[pl.ds(base, b_per_w)], idx_v)
        pltpu.sync_copy(tgt_h.at[pl.ds(base, b_per_w)], tgt_v)
        pltpu.sync_copy(lse_h, lse_v)

        gsems = (gsem0, gsem1)
        osems = (osem0, osem1)
        iot = lax.iota(jnp.int32, LANES)
        gcp = [None] * n_ch
        ocp = [None] * n_ch
        tail_offs = []
        o = 0
        while o + LANES < tail:
            tail_offs.append(o)
            o += LANES
        tail_offs.append(tail - LANES)

        def start_gather(c):
            p = c % 2
            gcp[c] = pltpu.async_copy(
                table_h.at[idx_v.at[pl.ds(c * CH, CH)]],
                rows_v.at[pl.ds(p * CH, CH)], gsems[p])

        start_gather(0)
        acc = jnp.zeros((LANES,), jnp.float32)
        for c in range(n_ch):
            p = c % 2
            gcp[c].wait()
            if c + 1 < n_ch:
                if c >= 1:
                    for d in ocp[c - 1]:
                        d.wait()
                start_gather(c + 1)
            for g in range(CH // LANES):
                off = c * CH + g * LANES
                idx_vals = idx_v[pl.ds(off, LANES)]
                tgt_vals = tgt_v[pl.ds(off, LANES)]
                lse_g = plsc.load_gather(lse_v, [idx_vals])
                t_log = plsc.load_gather(
                    rows_v, [p * CH + g * LANES + iot, tgt_vals])
                acc = acc + (lse_g - t_log)

            def repack_row(r, _):
                for o in tail_offs:
                    tails_v[p * CH + r, pl.ds(o, LANES)] = (
                        rows_v[p * CH + r, pl.ds(tail0 + o, LANES)])
                return 0
            lax.fori_loop(0, CH, repack_row, 0, unroll=False)

            r0 = base + c * CH
            cps = []
            for j in range(n_full):
                cps.append(pltpu.async_copy(
                    rows_v.at[pl.ds(p * CH, CH), pl.ds(j * 128, 128)],
                    out_h.at[pl.ds(r0, CH), pl.ds(j * 128, 128)], osems[p]))
            cps.append(pltpu.async_copy(
                tails_v.at[pl.ds(p * CH, CH)],
                out_h.at[pl.ds(r0, CH), pl.ds(tail0, tail)], osems[p]))
            ocp[c] = cps
        for d in ocp[n_ch - 2]:
            d.wait()
        for d in ocp[n_ch - 1]:
            d.wait()
        acc_v[...] = acc
        pltpu.sync_copy(acc_v, part_h.at[pl.ds(wid * LANES, LANES)])

    return pl.kernel(
        body,
        out_type=[
            jax.ShapeDtypeStruct((N, C), jnp.float32),
            jax.ShapeDtypeStruct((NW * LANES,), jnp.float32),
        ],
        mesh=mesh,
        compiler_params=pltpu.CompilerParams(needs_layout_passes=False),
        scratch_types=[
            pltpu.VMEM((b_per_w,), jnp.int32),
            pltpu.VMEM((b_per_w,), jnp.int32),
            pltpu.VMEM((V,), jnp.float32),
            pltpu.VMEM((2 * CH, CP), jnp.float32),
            pltpu.VMEM((2 * CH, tail), jnp.float32),
            pltpu.VMEM((LANES,), jnp.float32),
            pltpu.SemaphoreType.DMA,
            pltpu.SemaphoreType.DMA,
            pltpu.SemaphoreType.DMA,
            pltpu.SemaphoreType.DMA,
        ],
    )


def kernel(idx, targets, table):
    V, C = table.shape
    CP = (C + 127) // 128 * 128
    Bb, Tt = idx.shape
    N = Bb * Tt
    idx_f = idx.reshape(N).astype(jnp.int32)
    tgt_f = targets.reshape(N).astype(jnp.int32)

    lse, table_p = pl.pallas_call(
        _lse_pad_body,
        out_shape=[
            jax.ShapeDtypeStruct((V, 1), jnp.float32),
            jax.ShapeDtypeStruct((V, CP), jnp.float32),
        ],
    )(table)
    lse_f = lse.reshape(V)

    NK = 4
    NSUB = N // NK
    sc = _sc_gather_fn(V, C, CP, NSUB, CH=32)
    outs, parts = [], []
    for k in range(NK):
        o, p = sc(table_p, idx_f[k * NSUB:(k + 1) * NSUB],
                  tgt_f[k * NSUB:(k + 1) * NSUB], lse_f)
        outs.append(o)
        parts.append(p)
    out = jnp.concatenate(outs, axis=0)
    part = jnp.concatenate(parts)

    loss = pl.pallas_call(
        functools.partial(_loss_body, inv_n=1.0 / N),
        out_shape=jax.ShapeDtypeStruct((1, 1), jnp.float32),
    )(part.reshape(NK * NW, LANES))[0, 0]

    return out, loss


# speedup vs baseline: 1.6193x; 1.6193x over previous
import functools

import jax
import jax.numpy as jnp
from jax import lax
from jax.experimental import pallas as pl
from jax.experimental.pallas import tpu as pltpu
from jax.experimental.pallas import tpu_sc as plsc

NC = 2
NS = 16
LANES = 16
NW = NC * NS


def _lse_pad_body(t_ref, lse_ref, tp_ref):
    x = t_ref[...]
    m = jnp.max(x, axis=1, keepdims=True)
    s = jnp.sum(jnp.exp(x - m), axis=1, keepdims=True)
    lse_ref[...] = m + jnp.log(s)
    tp_ref[...] = jnp.pad(x, ((0, 0), (0, tp_ref.shape[1] - x.shape[1])))


def _loss_body(p_ref, o_ref, *, inv_n):
    o_ref[...] = jnp.full((1, 1), inv_n, jnp.float32) * jnp.sum(p_ref[...])


def _transpose_body(x_ref, o_ref, *, C):
    o_ref[...] = x_ref[...][:, :C].T


def _transpose_body_acc(x_ref, prev_ref, o_ref, *, C):
    o_ref[...] = x_ref[...][:, :C].T


def _sc_gather_fn(V, C, CP, N, CH):
    b_per_w = N // NW
    n_ch = b_per_w // CH
    mesh = plsc.VectorSubcoreMesh(core_axis_name="c", subcore_axis_name="s")

    def body(table_h, idx_h, tgt_h, lse_h, out_h, part_h,
             idx_v, tgt_v, lse_v, rows_v, acc_v, gsem0, gsem1, osem0, osem1):
        cid = lax.axis_index("c")
        sid = lax.axis_index("s")
        wid = sid * NC + cid
        base = wid * b_per_w
        pltpu.sync_copy(idx_h.at[pl.ds(base, b_per_w)], idx_v)
        pltpu.sync_copy(tgt_h.at[pl.ds(base, b_per_w)], tgt_v)
        pltpu.sync_copy(lse_h, lse_v)

        gsems = (gsem0, gsem1)
        osems = (osem0, osem1)
        iot = lax.iota(jnp.int32, LANES)
        gcp = [None] * n_ch
        ocp = [None] * n_ch

        def start_gather(c):
            p = c % 2
            gcp[c] = pltpu.async_copy(
                table_h.at[idx_v.at[pl.ds(c * CH, CH)]],
                rows_v.at[pl.ds(p * CH, CH)], gsems[p])

        start_gather(0)
        acc = jnp.zeros((LANES,), jnp.float32)
        for c in range(n_ch):
            p = c % 2
            gcp[c].wait()
            if c + 1 < n_ch:
                if c >= 1:
                    ocp[c - 1].wait()
                start_gather(c + 1)
            for g in range(CH // LANES):
                off = c * CH + g * LANES
                idx_vals = idx_v[pl.ds(off, LANES)]
                tgt_vals = tgt_v[pl.ds(off, LANES)]
                lse_g = plsc.load_gather(lse_v, [idx_vals])
                t_log = plsc.load_gather(
                    rows_v, [p * CH + g * LANES + iot, tgt_vals])
                acc = acc + (lse_g - t_log)
            ocp[c] = pltpu.async_copy(
                rows_v.at[pl.ds(p * CH, CH)],
                out_h.at[pl.ds(base + c * CH, CH)], osems[p])
        ocp[n_ch - 2].wait()
        ocp[n_ch - 1].wait()
        acc_v[...] = acc
        pltpu.sync_copy(acc_v, part_h.at[pl.ds(wid * LANES, LANES)])

    return pl.kernel(
        body,
        out_type=[
            jax.ShapeDtypeStruct((N, CP), jnp.float32),
            jax.ShapeDtypeStruct((NW * LANES,), jnp.float32),
        ],
        mesh=mesh,
        compiler_params=pltpu.CompilerParams(needs_layout_passes=False),
        scratch_types=[
            pltpu.VMEM((b_per_w,), jnp.int32),
            pltpu.VMEM((b_per_w,), jnp.int32),
            pltpu.VMEM((V,), jnp.float32),
            pltpu.VMEM((2 * CH, CP), jnp.float32),
            pltpu.VMEM((LANES,), jnp.float32),
            pltpu.SemaphoreType.DMA,
            pltpu.SemaphoreType.DMA,
            pltpu.SemaphoreType.DMA,
            pltpu.SemaphoreType.DMA,
        ],
    )


def kernel(idx, targets, table):
    V, C = table.shape
    CP = (C + 127) // 128 * 128
    Bb, Tt = idx.shape
    N = Bb * Tt
    idx_f = idx.reshape(N).astype(jnp.int32)
    tgt_f = targets.reshape(N).astype(jnp.int32)

    lse, table_p = pl.pallas_call(
        _lse_pad_body,
        out_shape=[
            jax.ShapeDtypeStruct((V, 1), jnp.float32),
            jax.ShapeDtypeStruct((V, CP), jnp.float32),
        ],
    )(table)
    lse_f = lse.reshape(V)

    NK = 2
    NSUB = N // NK
    BT = 512
    sc = _sc_gather_fn(V, C, CP, NSUB, CH=32)

    chunks, parts = [], []
    for k in range(NK):
        o, p = sc(table_p, idx_f[k * NSUB:(k + 1) * NSUB],
                  tgt_f[k * NSUB:(k + 1) * NSUB], lse_f)
        chunks.append(o)
        parts.append(p)

    outT = pl.pallas_call(
        functools.partial(_transpose_body, C=C),
        grid=(NSUB // BT,),
        in_specs=[pl.BlockSpec((BT, CP), lambda j: (j, 0))],
        out_specs=pl.BlockSpec((C, BT), lambda j: (0, j)),
        out_shape=jax.ShapeDtypeStruct((C, N), jnp.float32),
    )(chunks[0])
    for k in range(1, NK):
        outT = pl.pallas_call(
            functools.partial(_transpose_body_acc, C=C),
            grid=(NSUB // BT,),
            in_specs=[
                pl.BlockSpec((BT, CP), lambda j: (j, 0)),
                pl.BlockSpec(memory_space=pl.ANY),
            ],
            out_specs=pl.BlockSpec(
                (C, BT), lambda j, _k=k: (0, _k * (NSUB // BT) + j)),
            out_shape=jax.ShapeDtypeStruct((C, N), jnp.float32),
            input_output_aliases={1: 0},
        )(chunks[k], outT)
    out = outT.T

    part = jnp.concatenate(parts)
    loss = pl.pallas_call(
        functools.partial(_loss_body, inv_n=1.0 / N),
        out_shape=jax.ShapeDtypeStruct((1, 1), jnp.float32),
    )(part.reshape(NK * NW, LANES))[0, 0]

    return out, loss


# speedup vs baseline: 1.6408x; 1.0133x over previous
import functools

import jax
import jax.numpy as jnp
from jax import lax
from jax.experimental import pallas as pl
from jax.experimental.pallas import tpu as pltpu
from jax.experimental.pallas import tpu_sc as plsc

NC = 2
NS = 16
LANES = 16
NW = NC * NS


def _lse_pad_body(t_ref, lse_ref, tp_ref):
    x = t_ref[...]
    m = jnp.max(x, axis=1, keepdims=True)
    s = jnp.sum(jnp.exp(x - m), axis=1, keepdims=True)
    lse_ref[...] = m + jnp.log(s)
    tp_ref[...] = jnp.pad(x, ((0, 0), (0, tp_ref.shape[1] - x.shape[1])))


def _loss_body(p_ref, o_ref, *, inv_n):
    o_ref[...] = jnp.full((1, 1), inv_n, jnp.float32) * jnp.sum(p_ref[...])


def _sc_gather_fn(V, C, CP, N, CH):
    b_per_w = N // NW
    n_ch = b_per_w // CH
    mesh = plsc.VectorSubcoreMesh(core_axis_name="c", subcore_axis_name="s")

    def body(table_h, idx_h, tgt_h, lse_h, out_h, part_h,
             idx_v, tgt_v, lse_v, rows_v, acc_v,
             gsem0, gsem1, osem0, osem1, tsem, lsem):
        cid = lax.axis_index("c")
        sid = lax.axis_index("s")
        wid = sid * NC + cid
        base = wid * b_per_w
        pltpu.sync_copy(idx_h.at[pl.ds(base, b_per_w)], idx_v)
        tcp = pltpu.async_copy(tgt_h.at[pl.ds(base, b_per_w)], tgt_v, tsem)
        lcp = pltpu.async_copy(lse_h, lse_v, lsem)

        gsems = (gsem0, gsem1)
        osems = (osem0, osem1)
        iot = lax.iota(jnp.int32, LANES)
        gcp = [None] * n_ch
        ocp = [None] * n_ch

        def start_gather(c):
            p = c % 2
            gcp[c] = pltpu.async_copy(
                table_h.at[idx_v.at[pl.ds(c * CH, CH)]], rows_v.at[p], gsems[p])

        start_gather(0)
        tcp.wait()
        lcp.wait()
        acc = jnp.zeros((LANES,), jnp.float32)
        for c in range(n_ch):
            p = c % 2
            gcp[c].wait()
            if c + 1 < n_ch:
                if c >= 1:
                    ocp[c - 1].wait()
                start_gather(c + 1)
            for g in range(CH // LANES):
                off = c * CH + g * LANES
                idx_vals = idx_v[pl.ds(off, LANES)]
                tgt_vals = tgt_v[pl.ds(off, LANES)]
                lse_g = plsc.load_gather(lse_v, [idx_vals])
                t_log = plsc.load_gather(rows_v.at[p], [iot + g * LANES, tgt_vals])
                acc = acc + (lse_g - t_log)
            ocp[c] = pltpu.async_copy(
                rows_v.at[p], out_h.at[pl.ds(base + c * CH, CH)], osems[p])
        ocp[n_ch - 2].wait()
        ocp[n_ch - 1].wait()
        acc_v[...] = acc
        pltpu.sync_copy(acc_v, part_h.at[pl.ds(wid * LANES, LANES)])

    return pl.kernel(
        body,
        out_type=[
            jax.ShapeDtypeStruct((N, CP), jnp.float32),
            jax.ShapeDtypeStruct((NW * LANES,), jnp.float32),
        ],
        mesh=mesh,
        compiler_params=pltpu.CompilerParams(needs_layout_passes=False),
        scratch_types=[
            pltpu.VMEM((b_per_w,), jnp.int32),
            pltpu.VMEM((b_per_w,), jnp.int32),
            pltpu.VMEM((V,), jnp.float32),
            pltpu.VMEM((2, CH, CP), jnp.float32),
            pltpu.VMEM((LANES,), jnp.float32),
            pltpu.SemaphoreType.DMA,
            pltpu.SemaphoreType.DMA,
            pltpu.SemaphoreType.DMA,
            pltpu.SemaphoreType.DMA,
            pltpu.SemaphoreType.DMA,
            pltpu.SemaphoreType.DMA,
        ],
    )


def kernel(idx, targets, table):
    V, C = table.shape
    CP = (C + 127) // 128 * 128
    Bb, Tt = idx.shape
    N = Bb * Tt
    idx_f = idx.reshape(N).astype(jnp.int32)
    tgt_f = targets.reshape(N).astype(jnp.int32)

    lse, table_p = pl.pallas_call(
        _lse_pad_body,
        out_shape=[
            jax.ShapeDtypeStruct((V, 1), jnp.float32),
            jax.ShapeDtypeStruct((V, CP), jnp.float32),
        ],
    )(table)

    out_p, part = _sc_gather_fn(V, C, CP, N, CH=32)(
        table_p, idx_f, tgt_f, lse.reshape(V))
    out = out_p[:, :C]

    loss = pl.pallas_call(
        functools.partial(_loss_body, inv_n=1.0 / N),
        out_shape=jax.ShapeDtypeStruct((1, 1), jnp.float32),
    )(part.reshape(NW, LANES))[0, 0]

    return out, loss


# speedup vs baseline: 1.6508x; 1.0061x over previous
import functools

import jax
import jax.numpy as jnp
from jax import lax
from jax.experimental import pallas as pl
from jax.experimental.pallas import tpu as pltpu
from jax.experimental.pallas import tpu_sc as plsc

NC = 2
NS = 16
LANES = 16
NW = NC * NS


def _lse_pad_body(t_ref, lse_ref, tp_ref):
    x = t_ref[...]
    m = jnp.max(x, axis=1, keepdims=True)
    s = jnp.sum(jnp.exp(x - m), axis=1, keepdims=True)
    lse_ref[...] = m + jnp.log(s)
    tp_ref[...] = jnp.pad(x, ((0, 0), (0, tp_ref.shape[1] - x.shape[1])))


def _loss_body(p_ref, o_ref, *, inv_n):
    o_ref[...] = jnp.full((1, 1), inv_n, jnp.float32) * jnp.sum(p_ref[...])


def _sc_gather_fn(V, C, CP, N, CH):
    b_per_w = N // NW
    n_ch = b_per_w // CH
    mesh = plsc.VectorSubcoreMesh(core_axis_name="c", subcore_axis_name="s")

    def body(table_h, idx_h, tgt_h, lse_h, out_h, part_h,
             idx_v, tgt_v, lse_v, rows_v, acc_v, gsem0, gsem1, osem0, osem1):
        cid = lax.axis_index("c")
        sid = lax.axis_index("s")
        wid = sid * NC + cid
        base = wid * b_per_w
        pltpu.sync_copy(idx_h.at[pl.ds(base, b_per_w)], idx_v)
        pltpu.sync_copy(tgt_h.at[pl.ds(base, b_per_w)], tgt_v)
        pltpu.sync_copy(lse_h, lse_v)

        gsems = (gsem0, gsem1)
        osems = (osem0, osem1)
        iot = lax.iota(jnp.int32, LANES)
        gcp = [None] * n_ch
        ocp = [None] * n_ch

        def start_gather(c):
            p = c % 2
            gcp[c] = pltpu.async_copy(
                table_h.at[idx_v.at[pl.ds(c * CH, CH)]], rows_v.at[p], gsems[p])

        start_gather(0)
        acc = jnp.zeros((LANES,), jnp.float32)
        for c in range(n_ch):
            p = c % 2
            gcp[c].wait()
            if c + 1 < n_ch:
                if c >= 1:
                    ocp[c - 1].wait()
                start_gather(c + 1)
            for g in range(CH // LANES):
                off = c * CH + g * LANES
                idx_vals = idx_v[pl.ds(off, LANES)]
                tgt_vals = tgt_v[pl.ds(off, LANES)]
                lse_g = plsc.load_gather(lse_v, [idx_vals])
                t_log = plsc.load_gather(rows_v.at[p], [iot + g * LANES, tgt_vals])
                acc = acc + (lse_g - t_log)
            ocp[c] = pltpu.async_copy(
                rows_v.at[p], out_h.at[pl.ds(base + c * CH, CH)], osems[p])
        ocp[n_ch - 2].wait()
        ocp[n_ch - 1].wait()
        acc_v[...] = acc
        pltpu.sync_copy(acc_v, part_h.at[pl.ds(wid * LANES, LANES)])

    return pl.kernel(
        body,
        out_type=[
            jax.ShapeDtypeStruct((N, CP), jnp.float32),
            jax.ShapeDtypeStruct((NW * LANES,), jnp.float32),
        ],
        mesh=mesh,
        compiler_params=pltpu.CompilerParams(needs_layout_passes=False),
        scratch_types=[
            pltpu.VMEM((b_per_w,), jnp.int32),
            pltpu.VMEM((b_per_w,), jnp.int32),
            pltpu.VMEM((V,), jnp.float32),
            pltpu.VMEM((2, CH, CP), jnp.float32),
            pltpu.VMEM((LANES,), jnp.float32),
            pltpu.SemaphoreType.DMA,
            pltpu.SemaphoreType.DMA,
            pltpu.SemaphoreType.DMA,
            pltpu.SemaphoreType.DMA,
        ],
    )


def kernel(idx, targets, table):
    V, C = table.shape
    CP = (C + 127) // 128 * 128
    Bb, Tt = idx.shape
    N = Bb * Tt
    idx_f = idx.reshape(N).astype(jnp.int32)
    tgt_f = targets.reshape(N).astype(jnp.int32)

    lse, table_p = pl.pallas_call(
        _lse_pad_body,
        out_shape=[
            jax.ShapeDtypeStruct((V, 1), jnp.float32),
            jax.ShapeDtypeStruct((V, CP), jnp.float32),
        ],
    )(table)

    out_p, part = _sc_gather_fn(V, C, CP, N, CH=32)(
        table_p, idx_f, tgt_f, lse.reshape(V))
    out = out_p[:, :C]

    loss = pl.pallas_call(
        functools.partial(_loss_body, inv_n=1.0 / N),
        out_shape=jax.ShapeDtypeStruct((1, 1), jnp.float32),
    )(part.reshape(NW, LANES))[0, 0]

    return out, loss
